# baseline (speedup 1.0000x reference)
import functools

import jax
import jax.numpy as jnp
from jax import lax
from jax.experimental import pallas as pl
from jax.experimental.pallas import tpu as pltpu
from jax.experimental.pallas import tpu_sc as plsc

B, F, V, D = 4096, 26, 100000, 32
NC, NS = 2, 16
NW = NC * NS
BPW = B // NW
LANES = 16
NG = BPW // LANES
FGROUPS = [(0, 7), (7, 14), (14, 20), (20, 26)]
KG = len(FGROUPS)

_mesh = plsc.VectorSubcoreMesh(core_axis_name="c", subcore_axis_name="s",
                               num_cores=NC, num_subcores=NS)
_params = pltpu.CompilerParams(needs_layout_passes=False,
                               use_tc_tiling_on_sc=False)


def _make_group(f0, f1, with_lin):
    nf = f1 - f0

    def body(*args):
        if with_lin:
            (idxT, emb2, lin, s_out, q_out, lb_out,
             idx_v, rowsT_v, lin_v, s_v, q_v, lb_v, emb_sem, lin_sem) = args
        else:
            (idxT, emb2, s_out, q_out,
             idx_v, rowsT_v, s_v, q_v, emb_sem) = args
        wid = lax.axis_index("s") * NC + lax.axis_index("c")
        base = wid * BPW

        if with_lin:
            pltpu.sync_copy(idxT.at[:, pl.ds(base, BPW)], idx_v)
            for f in range(F):
                pltpu.async_copy(lin.at[f].at[idx_v.at[f]],
                                 lin_v.at[f], lin_sem)
        else:
            pltpu.sync_copy(idxT.at[pl.ds(f0, nf), pl.ds(base, BPW)], idx_v)

        def fire_emb(t, carry):
            f = lax.shift_right_logical(t, 5)
            pltpu.async_copy(emb2.at[t].at[idx_v.at[f]],
                             rowsT_v.at[t], emb_sem)
            return carry

        lax.fori_loop(0, nf * D, fire_emb, 0)

        if with_lin:
            pltpu.make_async_copy(lin.at[:, pl.ds(0, BPW)], lin_v,
                                  lin_sem).wait()
        pltpu.make_async_copy(emb2.at[:, pl.ds(0, BPW)], rowsT_v,
                              emb_sem).wait()

        def group_body(c, carry):
            col = c * LANES

            def dim_body(d, qacc):
                s = jnp.zeros((LANES,), jnp.float32)
                q = jnp.zeros((LANES,), jnp.float32)
                for f in range(nf):
                    v = rowsT_v[f * D + d, pl.ds(col, LANES)]
                    s = s + v
                    q = q + v * v
                s_v[d, pl.ds(col, LANES)] = s
                return qacc + q

            qacc = lax.fori_loop(0, D, dim_body,
                                 jnp.zeros((LANES,), jnp.float32))
            q_v[0, pl.ds(col, LANES)] = qacc
            if with_lin:
                lacc = jnp.zeros((LANES,), jnp.float32)
                for f in range(F):
                    lacc = lacc + lin_v[f, pl.ds(col, LANES)]
                lb_v[0, pl.ds(col, LANES)] = lacc
            return carry

        lax.fori_loop(0, NG, group_body, 0)

        pltpu.sync_copy(s_v, s_out.at[wid])
        pltpu.sync_copy(q_v, q_out.at[wid])
        if with_lin:
            pltpu.sync_copy(lb_v, lb_out.at[wid])

    out_types = [jax.ShapeDtypeStruct((NW, D, BPW), jnp.float32),
                 jax.ShapeDtypeStruct((NW, 1, BPW), jnp.float32)]
    scratch = [pltpu.VMEM(((F if with_lin else nf), BPW), jnp.int32),
               pltpu.VMEM((nf * D, BPW), jnp.float32),
               pltpu.VMEM((D, BPW), jnp.float32),
               pltpu.VMEM((1, BPW), jnp.float32),
               pltpu.SemaphoreType.DMA]
    if with_lin:
        out_types.append(jax.ShapeDtypeStruct((NW, 1, BPW), jnp.float32))
        scratch = ([scratch[0], scratch[1],
                    pltpu.VMEM((F, BPW), jnp.float32),
                    scratch[2], scratch[3],
                    pltpu.VMEM((1, BPW), jnp.float32),
                    scratch[4], pltpu.SemaphoreType.DMA])
    return functools.partial(
        pl.kernel, out_type=tuple(out_types), mesh=_mesh,
        compiler_params=_params, scratch_types=scratch)(body)


def _combine_body(s0, s1, s2, s3, q0, q1, q2, q3, lb, bias16, out,
                  sv, qv, lbv, bv, ov):
    wid = lax.axis_index("s") * NC + lax.axis_index("c")
    for i, s in enumerate((s0, s1, s2, s3)):
        pltpu.sync_copy(s.at[wid], sv.at[i])
    for i, q in enumerate((q0, q1, q2, q3)):
        pltpu.sync_copy(q.at[wid], qv.at[i])
    pltpu.sync_copy(lb.at[wid], lbv)
    pltpu.sync_copy(bias16, bv)

    def group_body(c, carry):
        col = c * LANES

        def dim_body(d, inter):
            s = sv[0, d, pl.ds(col, LANES)]
            for i in range(1, KG):
                s = s + sv[i, d, pl.ds(col, LANES)]
            return inter + s * s

        inter = lax.fori_loop(0, D, dim_body,
                              jnp.zeros((LANES,), jnp.float32))
        qt = qv[0, 0, pl.ds(col, LANES)]
        for i in range(1, KG):
            qt = qt + qv[i, 0, pl.ds(col, LANES)]
        ov[0, pl.ds(col, LANES)] = (0.5 * (inter - qt)
                                    + lbv[0, pl.ds(col, LANES)] + bv[...])
        return carry

    lax.fori_loop(0, NG, group_body, 0)
    pltpu.sync_copy(ov, out.at[wid])


_combine = functools.partial(
    pl.kernel,
    out_type=jax.ShapeDtypeStruct((NW, 1, BPW), jnp.float32),
    mesh=_mesh, compiler_params=_params,
    scratch_types=[
        pltpu.VMEM((KG, D, BPW), jnp.float32),
        pltpu.VMEM((KG, 1, BPW), jnp.float32),
        pltpu.VMEM((1, BPW), jnp.float32),
        pltpu.VMEM((LANES,), jnp.float32),
        pltpu.VMEM((1, BPW), jnp.float32),
    ])(_combine_body)

_group_kerns = [_make_group(f0, f1, with_lin=(g == 0))
                for g, (f0, f1) in enumerate(FGROUPS)]


def kernel(indices, embed_tables, lin_tables, bias):
    idxT = indices.T
    bias16 = jnp.broadcast_to(bias.astype(jnp.float32), (LANES,))
    parts = []
    for g, (f0, f1) in enumerate(FGROUPS):
        nf = f1 - f0
        emb_g = (embed_tables[f0:f1].transpose(0, 2, 1)
                 .reshape(nf * D, V))
        if g == 0:
            parts.append(_group_kerns[g](idxT, emb_g, lin_tables))
        else:
            parts.append(_group_kerns[g](idxT, emb_g))
    s_parts = [p[0] for p in parts]
    q_parts = [p[1] for p in parts]
    lb = parts[0][2]
    out = _combine(*s_parts, *q_parts, lb, bias16)
    return out.reshape(B)
